# baseline (speedup 1.0000x reference)
import jax
import jax.numpy as jnp
from jax import lax
from jax.experimental import pallas as pl
from jax.experimental.pallas import tpu as pltpu
from jax.experimental.pallas import tpu_sc as plsc

_NUM_FIELDS = 26
_VOCAB = 100000
_EMBED_DIM = 32
_BATCH = 16384

_NW = 32
_HALF = _BATCH // 2
_VEC = 16
_SPLIT = 50048
_VB = _VOCAB - _SPLIT


def _gather_body(xt_hbm, tabt_hbm, outt_hbm, src_a, src_b, idx_v, out_v,
                 sem_a, sem_b, sem_i, sem_o):
  cid = lax.axis_index("c")
  sid = lax.axis_index("s")
  d = sid * 2 + cid

  def fire_a(f):
    pltpu.async_copy(tabt_hbm.at[f, d, pl.ds(0, _SPLIT)], src_a, sem_a)

  def fire_b(f):
    pltpu.async_copy(tabt_hbm.at[f, d, pl.ds(_SPLIT, _VB)], src_b, sem_b)

  def wait_a():
    pltpu.make_async_copy(
        tabt_hbm.at[0, 0, pl.ds(0, _SPLIT)], src_a, sem_a).wait()

  def wait_b():
    pltpu.make_async_copy(
        tabt_hbm.at[0, 0, pl.ds(_SPLIT, _VB)], src_b, sem_b).wait()

  def fire_idx(f, h, buf):
    pltpu.async_copy(
        xt_hbm.at[f, pl.ds(h * _HALF, _HALF)], idx_v.at[buf], sem_i.at[buf])

  def wait_idx(buf):
    pltpu.make_async_copy(
        xt_hbm.at[0, pl.ds(0, _HALF)], idx_v.at[buf], sem_i.at[buf]).wait()

  def fire_out(f, h):
    pltpu.async_copy(
        out_v, outt_hbm.at[f, d, pl.ds(h * _HALF, _HALF)], sem_o)

  def wait_out():
    pltpu.make_async_copy(
        out_v, outt_hbm.at[0, 0, pl.ds(0, _HALF)], sem_o).wait()

  iota = lax.iota(jnp.int32, _VEC)

  def pass_a(buf):
    @plsc.parallel_loop(0, _HALF // _VEC, unroll=4)
    def _(i):
      vec = idx_v[buf, pl.ds(i * _VEC, _VEC)]
      g = plsc.load_gather(src_a, [jnp.minimum(vec, _SPLIT - 1)])
      out_v[pl.ds(i * _VEC, _VEC)] = g

  def pass_b(buf):
    @plsc.parallel_loop(0, _HALF // _VEC, unroll=4)
    def _(i):
      vec = idx_v[buf, pl.ds(i * _VEC, _VEC)] - _SPLIT
      m = vec >= 0
      g = plsc.load_gather(src_b, [vec], mask=m)
      plsc.store_scatter(out_v, [i * _VEC + iota], g, mask=m)

  fire_a(0)
  fire_b(0)
  fire_idx(0, 0, 0)

  wait_idx(0)
  fire_idx(0, 1, 1)
  wait_a()
  pass_a(0)
  wait_b()
  pass_b(0)
  fire_out(0, 0)
  wait_idx(1)
  fire_idx(1, 0, 0)
  wait_out()
  pass_a(1)
  fire_a(1)
  pass_b(1)
  fire_b(1)
  fire_out(0, 1)

  def field_step(f, _):
    fnext = jnp.minimum(f + 1, _NUM_FIELDS - 1)
    wait_idx(0)
    fire_idx(f, 1, 1)
    wait_a()
    wait_out()
    pass_a(0)
    wait_b()
    pass_b(0)
    fire_out(f, 0)
    wait_idx(1)
    fire_idx(fnext, 0, 0)
    wait_out()
    pass_a(1)
    fire_a(fnext)
    pass_b(1)
    fire_b(fnext)
    fire_out(f, 1)
    return 0

  lax.fori_loop(1, _NUM_FIELDS, field_step, 0)

  wait_idx(0)
  wait_a()
  wait_b()
  wait_out()


@jax.jit
def kernel(x, tables):
  x_t = x.T.astype(jnp.int32)
  tab_t = tables.transpose(0, 2, 1)

  mesh = plsc.VectorSubcoreMesh(core_axis_name="c", subcore_axis_name="s")
  run = pl.kernel(
      _gather_body,
      mesh=mesh,
      out_type=jax.ShapeDtypeStruct(
          (_NUM_FIELDS, _EMBED_DIM, _BATCH), jnp.float32),
      scratch_types=[
          pltpu.VMEM((_SPLIT,), jnp.float32),
          pltpu.VMEM((_VB,), jnp.float32),
          pltpu.VMEM((2, _HALF), jnp.int32),
          pltpu.VMEM((_HALF,), jnp.float32),
          pltpu.SemaphoreType.DMA,
          pltpu.SemaphoreType.DMA,
          pltpu.SemaphoreType.DMA((2,)),
          pltpu.SemaphoreType.DMA,
      ],
      compiler_params=pltpu.CompilerParams(
          use_tc_tiling_on_sc=True, needs_layout_passes=False),
  )
  out_t = run(x_t, tab_t)
  return out_t.transpose(2, 0, 1)


# speedup vs baseline: 1.0659x; 1.0659x over previous
import jax
import jax.numpy as jnp
from jax import lax
from jax.experimental import pallas as pl
from jax.experimental.pallas import tpu as pltpu
from jax.experimental.pallas import tpu_sc as plsc

_NUM_FIELDS = 26
_VOCAB = 100000
_EMBED_DIM = 32
_BATCH = 16384

_NW = 32
_HALF = _BATCH // 2
_VEC = 16
_SPLIT = 50048
_VB = _VOCAB - _SPLIT


def _gather_body(xt_hbm, tabt_hbm, outt_hbm, src_a, src_b, idx_v, out_v,
                 x_sp, sem_a, sem_b, sem_i, sem_o, sem_x):
  cid = lax.axis_index("c")
  sid = lax.axis_index("s")
  d = sid * 2 + cid
  is_leader = sid == 0

  def fire_x(f, slot):
    @pl.when(is_leader)
    def _():
      pltpu.async_copy(xt_hbm.at[f], x_sp.at[slot], sem_x)

  def wait_x():
    @pl.when(is_leader)
    def _():
      pltpu.make_async_copy(xt_hbm.at[0], x_sp.at[0], sem_x).wait()

  def fire_a(f):
    pltpu.async_copy(tabt_hbm.at[f, d, pl.ds(0, _SPLIT)], src_a, sem_a)

  def fire_b(f):
    pltpu.async_copy(tabt_hbm.at[f, d, pl.ds(_SPLIT, _VB)], src_b, sem_b)

  def wait_a():
    pltpu.make_async_copy(
        tabt_hbm.at[0, 0, pl.ds(0, _SPLIT)], src_a, sem_a).wait()

  def wait_b():
    pltpu.make_async_copy(
        tabt_hbm.at[0, 0, pl.ds(_SPLIT, _VB)], src_b, sem_b).wait()

  def fire_idx(f, h, buf):
    pltpu.async_copy(
        x_sp.at[f % 2, pl.ds(h * _HALF, _HALF)], idx_v.at[buf],
        sem_i.at[buf])

  def wait_idx(buf):
    pltpu.make_async_copy(
        x_sp.at[0, pl.ds(0, _HALF)], idx_v.at[buf], sem_i.at[buf]).wait()

  def fire_out(f, h):
    pltpu.async_copy(
        out_v, outt_hbm.at[f, d, pl.ds(h * _HALF, _HALF)], sem_o)

  def wait_out():
    pltpu.make_async_copy(
        out_v, outt_hbm.at[0, 0, pl.ds(0, _HALF)], sem_o).wait()

  iota = lax.iota(jnp.int32, _VEC)

  def pass_a(buf):
    @plsc.parallel_loop(0, _HALF // _VEC, unroll=4)
    def _(i):
      vec = idx_v[buf, pl.ds(i * _VEC, _VEC)]
      g = plsc.load_gather(src_a, [jnp.minimum(vec, _SPLIT - 1)])
      out_v[pl.ds(i * _VEC, _VEC)] = g

  def pass_b(buf):
    @plsc.parallel_loop(0, _HALF // _VEC, unroll=4)
    def _(i):
      vec = idx_v[buf, pl.ds(i * _VEC, _VEC)] - _SPLIT
      m = vec >= 0
      g = plsc.load_gather(src_b, [vec], mask=m)
      plsc.store_scatter(out_v, [i * _VEC + iota], g, mask=m)

  fire_a(0)
  fire_b(0)
  fire_x(0, 0)
  fire_x(1, 1)
  wait_x()
  plsc.subcore_barrier()
  fire_idx(0, 0, 0)

  wait_idx(0)
  fire_idx(0, 1, 1)
  wait_a()
  pass_a(0)
  wait_b()
  pass_b(0)
  fire_out(0, 0)
  wait_idx(1)
  wait_x()
  plsc.subcore_barrier()
  fire_x(2, 0)
  fire_idx(1, 0, 0)
  wait_out()
  pass_a(1)
  fire_a(1)
  pass_b(1)
  fire_b(1)
  fire_out(0, 1)

  def field_step(f, _):
    fnext = jnp.minimum(f + 1, _NUM_FIELDS - 1)
    wait_idx(0)
    fire_idx(f, 1, 1)
    wait_a()
    wait_out()
    pass_a(0)
    wait_b()
    pass_b(0)
    fire_out(f, 0)
    wait_idx(1)
    wait_x()
    plsc.subcore_barrier()
    fire_x(jnp.minimum(f + 2, _NUM_FIELDS - 1), f % 2)
    fire_idx(fnext, 0, 0)
    wait_out()
    pass_a(1)
    fire_a(fnext)
    pass_b(1)
    fire_b(fnext)
    fire_out(f, 1)
    return 0

  lax.fori_loop(1, _NUM_FIELDS, field_step, 0)

  wait_idx(0)
  wait_x()
  wait_a()
  wait_b()
  wait_out()


@jax.jit
def kernel(x, tables):
  x_t = x.T.astype(jnp.int32)
  tab_t = tables.transpose(0, 2, 1)

  mesh = plsc.VectorSubcoreMesh(core_axis_name="c", subcore_axis_name="s")
  run = pl.kernel(
      _gather_body,
      mesh=mesh,
      out_type=jax.ShapeDtypeStruct(
          (_NUM_FIELDS, _EMBED_DIM, _BATCH), jnp.float32),
      scratch_types=[
          pltpu.VMEM((_SPLIT,), jnp.float32),
          pltpu.VMEM((_VB,), jnp.float32),
          pltpu.VMEM((2, _HALF), jnp.int32),
          pltpu.VMEM((_HALF,), jnp.float32),
          pltpu.VMEM_SHARED((2, _BATCH), jnp.int32),
          pltpu.SemaphoreType.DMA,
          pltpu.SemaphoreType.DMA,
          pltpu.SemaphoreType.DMA((2,)),
          pltpu.SemaphoreType.DMA,
          pltpu.SemaphoreType.DMA,
      ],
      compiler_params=pltpu.CompilerParams(
          use_tc_tiling_on_sc=True, needs_layout_passes=False),
  )
  out_t = run(x_t, tab_t)
  return out_t.transpose(2, 0, 1)


# speedup vs baseline: 1.1004x; 1.0324x over previous
import jax
import jax.numpy as jnp
from jax import lax
from jax.experimental import pallas as pl
from jax.experimental.pallas import tpu as pltpu
from jax.experimental.pallas import tpu_sc as plsc

_NUM_FIELDS = 26
_VOCAB = 100000
_EMBED_DIM = 32
_BATCH = 16384

_NW = 32
_HALF = _BATCH // 2
_VEC = 16
_SPLIT = 50048
_VB = _VOCAB - _SPLIT


def _gather_body(xt_hbm, tabt_hbm, outt_hbm, src_a, src_b, idx_v, out_v,
                 x_sp, sem_a, sem_b, sem_i, sem_o, sem_x):
  cid = lax.axis_index("c")
  sid = lax.axis_index("s")
  d = sid * 2 + cid
  is_leader = sid == 0

  def fire_x(f, slot):
    @pl.when(is_leader)
    def _():
      pltpu.async_copy(xt_hbm.at[f], x_sp.at[slot], sem_x)

  def wait_x():
    @pl.when(is_leader)
    def _():
      pltpu.make_async_copy(xt_hbm.at[0], x_sp.at[0], sem_x).wait()

  def fire_a(f):
    pltpu.async_copy(tabt_hbm.at[f, d, pl.ds(0, _SPLIT)], src_a, sem_a)

  def fire_b(f):
    pltpu.async_copy(tabt_hbm.at[f, d, pl.ds(_SPLIT, _VB)], src_b, sem_b)

  def wait_a():
    pltpu.make_async_copy(
        tabt_hbm.at[0, 0, pl.ds(0, _SPLIT)], src_a, sem_a).wait()

  def wait_b():
    pltpu.make_async_copy(
        tabt_hbm.at[0, 0, pl.ds(_SPLIT, _VB)], src_b, sem_b).wait()

  def fire_idx(f, h, buf):
    pltpu.async_copy(
        x_sp.at[f % 2, pl.ds(h * _HALF, _HALF)], idx_v.at[buf],
        sem_i.at[buf])

  def wait_idx(buf):
    pltpu.make_async_copy(
        x_sp.at[0, pl.ds(0, _HALF)], idx_v.at[buf], sem_i.at[buf]).wait()

  def fire_out(f, h):
    pltpu.async_copy(
        out_v, outt_hbm.at[f, d, pl.ds(h * _HALF, _HALF)], sem_o)

  def wait_out():
    pltpu.make_async_copy(
        out_v, outt_hbm.at[0, 0, pl.ds(0, _HALF)], sem_o).wait()

  iota = lax.iota(jnp.int32, _VEC)

  def pass_a(buf):
    @plsc.parallel_loop(0, _HALF // _VEC, unroll=4)
    def _(i):
      vec = idx_v[buf, pl.ds(i * _VEC, _VEC)]
      g = plsc.load_gather(src_a, [jnp.minimum(vec, _SPLIT - 1)])
      out_v[pl.ds(i * _VEC, _VEC)] = g

  def pass_b(buf):
    @plsc.parallel_loop(0, _HALF // _VEC, unroll=4)
    def _(i):
      vec = idx_v[buf, pl.ds(i * _VEC, _VEC)] - _SPLIT
      m = vec >= 0
      g = plsc.load_gather(src_b, [vec], mask=m)
      plsc.store_scatter(out_v, [i * _VEC + iota], g, mask=m)

  fire_a(0)
  fire_b(0)
  fire_x(0, 0)
  fire_x(1, 1)
  wait_x()
  plsc.subcore_barrier()
  fire_idx(0, 0, 0)

  wait_idx(0)
  fire_idx(0, 1, 1)
  wait_a()
  pass_a(0)
  wait_b()
  pass_b(0)
  fire_out(0, 0)
  wait_idx(1)
  wait_out()
  pass_a(1)
  fire_a(1)
  wait_x()
  plsc.subcore_barrier()
  fire_x(2, 0)
  fire_idx(1, 0, 0)
  pass_b(1)
  fire_b(1)
  fire_out(0, 1)

  def field_step(f, _):
    fnext = jnp.minimum(f + 1, _NUM_FIELDS - 1)
    wait_idx(0)
    fire_idx(f, 1, 1)
    wait_a()
    wait_out()
    pass_a(0)
    wait_b()
    pass_b(0)
    fire_out(f, 0)
    wait_idx(1)
    wait_out()
    pass_a(1)
    fire_a(fnext)
    wait_x()
    plsc.subcore_barrier()
    fire_x(jnp.minimum(f + 2, _NUM_FIELDS - 1), f % 2)
    fire_idx(fnext, 0, 0)
    pass_b(1)
    fire_b(fnext)
    fire_out(f, 1)
    return 0

  lax.fori_loop(1, _NUM_FIELDS, field_step, 0)

  wait_idx(0)
  wait_x()
  wait_a()
  wait_b()
  wait_out()


@jax.jit
def kernel(x, tables):
  x_t = x.T.astype(jnp.int32)
  tab_t = tables.transpose(0, 2, 1)

  mesh = plsc.VectorSubcoreMesh(core_axis_name="c", subcore_axis_name="s")
  run = pl.kernel(
      _gather_body,
      mesh=mesh,
      out_type=jax.ShapeDtypeStruct(
          (_NUM_FIELDS, _EMBED_DIM, _BATCH), jnp.float32),
      scratch_types=[
          pltpu.VMEM((_SPLIT,), jnp.float32),
          pltpu.VMEM((_VB,), jnp.float32),
          pltpu.VMEM((2, _HALF), jnp.int32),
          pltpu.VMEM((_HALF,), jnp.float32),
          pltpu.VMEM_SHARED((2, _BATCH), jnp.int32),
          pltpu.SemaphoreType.DMA,
          pltpu.SemaphoreType.DMA,
          pltpu.SemaphoreType.DMA((2,)),
          pltpu.SemaphoreType.DMA,
          pltpu.SemaphoreType.DMA,
      ],
      compiler_params=pltpu.CompilerParams(
          use_tc_tiling_on_sc=True, needs_layout_passes=False),
  )
  out_t = run(x_t, tab_t)
  return out_t.transpose(2, 0, 1)
